# baseline (speedup 1.0000x reference)
import dataclasses
import functools

import jax
import jax.numpy as jnp
from jax import lax
from jax.experimental import pallas as pl
from jax.experimental.pallas import tpu as pltpu
from jax.experimental.pallas import tpu_sc as plsc

N_NODES = 10000
N_PAD = 10112
N_DEG = 10240
N_EDGES = 320000
D = 128

NUM_CORES = 2
NUM_SUBCORES = 16
EDGES_PER_SUBCORE = N_EDGES // NUM_SUBCORES
BATCH = 80
FAST_CORE = 0
CHUNKS_F = 183
CHUNKS_S = 67
EDGES_F = CHUNKS_F * BATCH
EDGES_S = CHUNKS_S * BATCH
ROWS_PER_SUBCORE = N_PAD // NUM_SUBCORES
LANES = 16

_MESH = plsc.VectorSubcoreMesh(core_axis_name="c", subcore_axis_name="s")

_SC_PARAMS = pltpu.CompilerParams()
if "needs_layout_passes" in pltpu.CompilerParams.__dataclass_fields__:
    _SC_PARAMS = dataclasses.replace(_SC_PARAMS, needs_layout_passes=False)


@functools.partial(
    pl.kernel,
    out_type=jax.ShapeDtypeStruct((NUM_CORES, N_DEG // D, D), jnp.float32),
    mesh=_MESH,
    scratch_types=[
        pltpu.VMEM((EDGES_PER_SUBCORE,), jnp.int32),
        pltpu.VMEM((N_DEG // D, D), jnp.float32),
        pltpu.VMEM((1, N_DEG // D), jnp.int32),
        pltpu.VMEM_SHARED((N_DEG // D, D), jnp.float32),
    ],
    compiler_params=_SC_PARAMS,
)
def _degree_kernel(dstf_hbm, dsts_hbm, ident_hbm, out_hbm, idx_v, deg_v,
                   ident_v, deg_shared):
    c = lax.axis_index("c")
    s = lax.axis_index("s")

    @pl.loop(0, N_DEG // D)
    def _(rr):
        @pl.loop(0, D // LANES)
        def _(cc):
            deg_v[rr, pl.ds(cc * LANES, LANES)] = jnp.zeros(
                (LANES,), jnp.float32)

    @pl.when(s < 5)
    def _():
        pltpu.sync_copy(deg_v.at[pl.ds(s * 16, 16)],
                        deg_shared.at[pl.ds(s * 16, 16)])

    pltpu.sync_copy(ident_hbm, ident_v)
    pltpu.sync_copy(dstf_hbm.at[s, 0], idx_v.at[pl.ds(0, EDGES_F)])
    pltpu.sync_copy(dsts_hbm.at[s, 0],
                    idx_v.at[pl.ds(EDGES_F, EDGES_S)])
    ones = jnp.full((LANES,), 1.0, jnp.float32)

    @pl.loop(0, EDGES_PER_SUBCORE // LANES)
    def _(i):
        idx = idx_v[pl.ds(i * LANES, LANES)]
        plsc.addupdate_scatter(deg_v, [idx >> 7, idx & 127], ones)

    plsc.subcore_barrier()

    pltpu.sync_copy(deg_v, deg_shared.at[ident_v.at[0]], add=True)

    plsc.subcore_barrier()

    @pl.when(s < 5)
    def _():
        pltpu.sync_copy(deg_shared.at[pl.ds(s * 16, 16)],
                        out_hbm.at[c, pl.ds(s * 16, 16)])


@functools.partial(
    pl.kernel,
    out_type=jax.ShapeDtypeStruct((NUM_CORES, N_PAD, D), jnp.float32),
    mesh=_MESH,
    scratch_types=[
        pltpu.VMEM((EDGES_F,), jnp.int32),
        pltpu.VMEM((EDGES_F,), jnp.int32),
        pltpu.VMEM((BATCH, D), jnp.float32),
        pltpu.VMEM((BATCH, D), jnp.float32),
        pltpu.VMEM_SHARED((N_PAD, D), jnp.float32),
        pltpu.SemaphoreType.DMA,
        pltpu.SemaphoreType.DMA,
    ],
    compiler_params=_SC_PARAMS,
)
def _aggregate_kernel(g_hbm, srcf_hbm, dstf_hbm, srcs_hbm, dsts_hbm,
                      zero_hbm, out_hbm,
                      src_v, dst_v, rows0_v, rows1_v, acc_shared,
                      sg0, sg1):
    c = lax.axis_index("c")
    s = lax.axis_index("s")
    row0 = s * ROWS_PER_SUBCORE

    @pl.when(c == FAST_CORE)
    def _():
        pltpu.sync_copy(g_hbm.at[pl.ds(row0, ROWS_PER_SUBCORE)],
                        acc_shared.at[pl.ds(row0, ROWS_PER_SUBCORE)])

    @pl.when(c != FAST_CORE)
    def _():
        pltpu.sync_copy(zero_hbm,
                        acc_shared.at[pl.ds(row0, ROWS_PER_SUBCORE)])

    def run(src_hbm, dst_hbm, nchunks):
        pltpu.sync_copy(src_hbm.at[s, 0],
                        src_v.at[pl.ds(0, nchunks * BATCH)])
        pltpu.sync_copy(dst_hbm.at[s, 0],
                        dst_v.at[pl.ds(0, nchunks * BATCH)])
        plsc.subcore_barrier()

        def gather(j, rows, sem):
            pltpu.async_copy(g_hbm.at[src_v.at[pl.ds(j * BATCH, BATCH)]],
                             rows, sem)

        def gather_wait(j, rows, sem):
            pltpu.make_async_copy(
                g_hbm.at[src_v.at[pl.ds(j * BATCH, BATCH)]],
                rows, sem).wait()

        def scatter(j, rows):
            pltpu.sync_copy(
                rows, acc_shared.at[dst_v.at[pl.ds(j * BATCH, BATCH)]],
                add=True)

        gather(0, rows0_v, sg0)

        @pl.loop(0, (nchunks - 1) // 2)
        def _(i):
            j = 2 * i
            gather(j + 1, rows1_v, sg1)
            gather_wait(j, rows0_v, sg0)
            scatter(j, rows0_v)
            gather(j + 2, rows0_v, sg0)
            gather_wait(j + 1, rows1_v, sg1)
            scatter(j + 1, rows1_v)

        gather_wait(nchunks - 1, rows0_v, sg0)
        scatter(nchunks - 1, rows0_v)

    @pl.when(c == FAST_CORE)
    def _():
        run(srcf_hbm, dstf_hbm, CHUNKS_F)

    @pl.when(c != FAST_CORE)
    def _():
        run(srcs_hbm, dsts_hbm, CHUNKS_S)

    plsc.subcore_barrier()
    pltpu.sync_copy(acc_shared.at[pl.ds(row0, ROWS_PER_SUBCORE)],
                    out_hbm.at[c, pl.ds(row0, ROWS_PER_SUBCORE)])


_BLOCK = 2000


def _matmul_body(x_ref, w_ref, b_ref, h_ref):
    h_ref[...] = lax.dot_general(
        x_ref[...], w_ref[...], (((1,), (1,)), ((), ())),
        preferred_element_type=jnp.float32) + b_ref[...]


def _matmul(x, w, b2d):
    return pl.pallas_call(
        _matmul_body,
        grid=(N_NODES // _BLOCK,),
        in_specs=[
            pl.BlockSpec((_BLOCK, D), lambda i: (i, 0)),
            pl.BlockSpec((D, D), lambda i: (0, 0)),
            pl.BlockSpec((1, D), lambda i: (0, 0)),
        ],
        out_specs=pl.BlockSpec((_BLOCK, D), lambda i: (i, 0)),
        out_shape=jax.ShapeDtypeStruct((N_NODES, D), jnp.float32),
    )(x, w, b2d)


def _scale_body(deg_ref, h_ref, g_ref, r_ref):
    r = lax.rsqrt(deg_ref[...] + 1.0)
    r_ref[...] = r
    g_ref[...] = h_ref[...] * r


def _scale(deg_col, h):
    return pl.pallas_call(
        _scale_body,
        grid=(N_NODES // _BLOCK,),
        in_specs=[
            pl.BlockSpec((_BLOCK, 1), lambda i: (i, 0)),
            pl.BlockSpec((_BLOCK, D), lambda i: (i, 0)),
        ],
        out_specs=[
            pl.BlockSpec((_BLOCK, D), lambda i: (i, 0)),
            pl.BlockSpec((_BLOCK, 1), lambda i: (i, 0)),
        ],
        out_shape=[
            jax.ShapeDtypeStruct((N_PAD, D), jnp.float32),
            jax.ShapeDtypeStruct((N_NODES, 1), jnp.float32),
        ],
    )(deg_col, h)


def _combine_body(p_ref, r_ref, o_ref):
    o_ref[...] = (p_ref[0] + p_ref[1]) * r_ref[...]


def _combine(partials, r):
    return pl.pallas_call(
        _combine_body,
        grid=(N_NODES // _BLOCK,),
        in_specs=[
            pl.BlockSpec((NUM_CORES, _BLOCK, D), lambda i: (0, i, 0)),
            pl.BlockSpec((_BLOCK, 1), lambda i: (i, 0)),
        ],
        out_specs=pl.BlockSpec((_BLOCK, D), lambda i: (i, 0)),
        out_shape=jax.ShapeDtypeStruct((N_NODES, D), jnp.float32),
    )(partials, r)


def kernel(x, edge_index, W, b):
    edges = edge_index.astype(jnp.int32)
    src, dst = edges[0], edges[1]
    nf = NUM_SUBCORES * EDGES_F
    src_f = src[:nf].reshape(NUM_SUBCORES, 1, EDGES_F)
    dst_f = dst[:nf].reshape(NUM_SUBCORES, 1, EDGES_F)
    src_s = src[nf:].reshape(NUM_SUBCORES, 1, EDGES_S)
    dst_s = dst[nf:].reshape(NUM_SUBCORES, 1, EDGES_S)
    ident = jnp.arange(N_DEG // D, dtype=jnp.int32).reshape(1, N_DEG // D)

    deg2 = _degree_kernel(dst_f, dst_s, ident)
    h = _matmul(x, W, b.reshape(1, D))
    deg_col = deg2[0].reshape(N_DEG)[:N_NODES, None]
    g, r = _scale(deg_col, h)
    zeros = jnp.zeros((ROWS_PER_SUBCORE, D), jnp.float32)
    partials = _aggregate_kernel(g, src_f, dst_f, src_s, dst_s, zeros)
    return _combine(partials, r)


# speedup vs baseline: 1.1894x; 1.1894x over previous
import dataclasses
import functools

import jax
import jax.numpy as jnp
from jax import lax
from jax.experimental import pallas as pl
from jax.experimental.pallas import tpu as pltpu
from jax.experimental.pallas import tpu_sc as plsc

N_NODES = 10000
N_PAD = 10112
N_DEG = 10240
N_EDGES = 320000
D = 128

NUM_CORES = 2
NUM_SUBCORES = 16
EDGES_PER_SUBCORE = N_EDGES // NUM_SUBCORES
BATCH = 80
FAST_CORE = 0
CHUNKS_F = 135
CHUNKS_S = 115
EDGES_F = CHUNKS_F * BATCH
EDGES_S = CHUNKS_S * BATCH
ROWS_PER_SUBCORE = N_PAD // NUM_SUBCORES
LANES = 16

_MESH = plsc.VectorSubcoreMesh(core_axis_name="c", subcore_axis_name="s")

_SC_PARAMS = pltpu.CompilerParams()
if "needs_layout_passes" in pltpu.CompilerParams.__dataclass_fields__:
    _SC_PARAMS = dataclasses.replace(_SC_PARAMS, needs_layout_passes=False)


@functools.partial(
    pl.kernel,
    out_type=jax.ShapeDtypeStruct((NUM_CORES, N_DEG // D, D), jnp.float32),
    mesh=_MESH,
    scratch_types=[
        pltpu.VMEM((EDGES_PER_SUBCORE,), jnp.int32),
        pltpu.VMEM((N_DEG // D, D), jnp.float32),
        pltpu.VMEM((1, N_DEG // D), jnp.int32),
        pltpu.VMEM_SHARED((N_DEG // D, D), jnp.float32),
    ],
    compiler_params=_SC_PARAMS,
)
def _degree_kernel(dstf_hbm, dsts_hbm, ident_hbm, out_hbm, idx_v, deg_v,
                   ident_v, deg_shared):
    c = lax.axis_index("c")
    s = lax.axis_index("s")

    @pl.loop(0, N_DEG // D)
    def _(rr):
        @pl.loop(0, D // LANES)
        def _(cc):
            deg_v[rr, pl.ds(cc * LANES, LANES)] = jnp.zeros(
                (LANES,), jnp.float32)

    @pl.when(s < 5)
    def _():
        pltpu.sync_copy(deg_v.at[pl.ds(s * 16, 16)],
                        deg_shared.at[pl.ds(s * 16, 16)])

    pltpu.sync_copy(ident_hbm, ident_v)
    pltpu.sync_copy(dstf_hbm.at[s, 0], idx_v.at[pl.ds(0, EDGES_F)])
    pltpu.sync_copy(dsts_hbm.at[s, 0],
                    idx_v.at[pl.ds(EDGES_F, EDGES_S)])
    ones = jnp.full((LANES,), 1.0, jnp.float32)

    @pl.loop(0, EDGES_PER_SUBCORE // LANES)
    def _(i):
        idx = idx_v[pl.ds(i * LANES, LANES)]
        plsc.addupdate_scatter(deg_v, [idx >> 7, idx & 127], ones)

    plsc.subcore_barrier()

    pltpu.sync_copy(deg_v, deg_shared.at[ident_v.at[0]], add=True)

    plsc.subcore_barrier()

    @pl.when(s < 5)
    def _():
        pltpu.sync_copy(deg_shared.at[pl.ds(s * 16, 16)],
                        out_hbm.at[c, pl.ds(s * 16, 16)])


@functools.partial(
    pl.kernel,
    out_type=jax.ShapeDtypeStruct((NUM_CORES, N_PAD, D), jnp.float32),
    mesh=_MESH,
    scratch_types=[
        pltpu.VMEM((EDGES_F,), jnp.int32),
        pltpu.VMEM((EDGES_F,), jnp.int32),
        pltpu.VMEM((BATCH, D), jnp.float32),
        pltpu.VMEM((BATCH, D), jnp.float32),
        pltpu.VMEM_SHARED((N_PAD, D), jnp.float32),
        pltpu.SemaphoreType.DMA,
        pltpu.SemaphoreType.DMA,
    ],
    compiler_params=_SC_PARAMS,
)
def _aggregate_kernel(g_hbm, srcf_hbm, dstf_hbm, srcs_hbm, dsts_hbm,
                      zero_hbm, out_hbm,
                      src_v, dst_v, rows0_v, rows1_v, acc_shared,
                      sg0, sg1):
    c = lax.axis_index("c")
    s = lax.axis_index("s")
    row0 = s * ROWS_PER_SUBCORE

    @pl.when(c == FAST_CORE)
    def _():
        pltpu.sync_copy(g_hbm.at[pl.ds(row0, ROWS_PER_SUBCORE)],
                        acc_shared.at[pl.ds(row0, ROWS_PER_SUBCORE)])

    @pl.when(c != FAST_CORE)
    def _():
        pltpu.sync_copy(zero_hbm,
                        acc_shared.at[pl.ds(row0, ROWS_PER_SUBCORE)])

    def run(src_hbm, dst_hbm, nchunks):
        pltpu.sync_copy(src_hbm.at[s, 0],
                        src_v.at[pl.ds(0, nchunks * BATCH)])
        pltpu.sync_copy(dst_hbm.at[s, 0],
                        dst_v.at[pl.ds(0, nchunks * BATCH)])
        plsc.subcore_barrier()

        def gather(j, rows, sem):
            pltpu.async_copy(g_hbm.at[src_v.at[pl.ds(j * BATCH, BATCH)]],
                             rows, sem)

        def gather_wait(j, rows, sem):
            pltpu.make_async_copy(
                g_hbm.at[src_v.at[pl.ds(j * BATCH, BATCH)]],
                rows, sem).wait()

        def scatter(j, rows):
            pltpu.sync_copy(
                rows, acc_shared.at[dst_v.at[pl.ds(j * BATCH, BATCH)]],
                add=True)

        gather(0, rows0_v, sg0)

        @pl.loop(0, (nchunks - 1) // 2)
        def _(i):
            j = 2 * i
            gather(j + 1, rows1_v, sg1)
            gather_wait(j, rows0_v, sg0)
            scatter(j, rows0_v)
            gather(j + 2, rows0_v, sg0)
            gather_wait(j + 1, rows1_v, sg1)
            scatter(j + 1, rows1_v)

        gather_wait(nchunks - 1, rows0_v, sg0)
        scatter(nchunks - 1, rows0_v)

    @pl.when(c == FAST_CORE)
    def _():
        run(srcf_hbm, dstf_hbm, CHUNKS_F)

    @pl.when(c != FAST_CORE)
    def _():
        run(srcs_hbm, dsts_hbm, CHUNKS_S)

    plsc.subcore_barrier()
    pltpu.sync_copy(acc_shared.at[pl.ds(row0, ROWS_PER_SUBCORE)],
                    out_hbm.at[c, pl.ds(row0, ROWS_PER_SUBCORE)])


_BLOCK = 2000


def _matmul_body(x_ref, w_ref, b_ref, h_ref):
    h_ref[...] = lax.dot_general(
        x_ref[...], w_ref[...], (((1,), (1,)), ((), ())),
        preferred_element_type=jnp.float32) + b_ref[...]


def _matmul(x, w, b2d):
    return pl.pallas_call(
        _matmul_body,
        grid=(N_NODES // _BLOCK,),
        in_specs=[
            pl.BlockSpec((_BLOCK, D), lambda i: (i, 0)),
            pl.BlockSpec((D, D), lambda i: (0, 0)),
            pl.BlockSpec((1, D), lambda i: (0, 0)),
        ],
        out_specs=pl.BlockSpec((_BLOCK, D), lambda i: (i, 0)),
        out_shape=jax.ShapeDtypeStruct((N_NODES, D), jnp.float32),
    )(x, w, b2d)


def _scale_body(deg_ref, h_ref, g_ref, r_ref):
    r = lax.rsqrt(deg_ref[...] + 1.0)
    r_ref[...] = r
    g_ref[...] = h_ref[...] * r


def _scale(deg_col, h):
    return pl.pallas_call(
        _scale_body,
        grid=(N_NODES // _BLOCK,),
        in_specs=[
            pl.BlockSpec((_BLOCK, 1), lambda i: (i, 0)),
            pl.BlockSpec((_BLOCK, D), lambda i: (i, 0)),
        ],
        out_specs=[
            pl.BlockSpec((_BLOCK, D), lambda i: (i, 0)),
            pl.BlockSpec((_BLOCK, 1), lambda i: (i, 0)),
        ],
        out_shape=[
            jax.ShapeDtypeStruct((N_PAD, D), jnp.float32),
            jax.ShapeDtypeStruct((N_NODES, 1), jnp.float32),
        ],
    )(deg_col, h)


def _combine_body(p_ref, r_ref, o_ref):
    o_ref[...] = (p_ref[0] + p_ref[1]) * r_ref[...]


def _combine(partials, r):
    return pl.pallas_call(
        _combine_body,
        grid=(N_NODES // _BLOCK,),
        in_specs=[
            pl.BlockSpec((NUM_CORES, _BLOCK, D), lambda i: (0, i, 0)),
            pl.BlockSpec((_BLOCK, 1), lambda i: (i, 0)),
        ],
        out_specs=pl.BlockSpec((_BLOCK, D), lambda i: (i, 0)),
        out_shape=jax.ShapeDtypeStruct((N_NODES, D), jnp.float32),
    )(partials, r)


def kernel(x, edge_index, W, b):
    edges = edge_index.astype(jnp.int32)
    src, dst = edges[0], edges[1]
    nf = NUM_SUBCORES * EDGES_F
    src_f = src[:nf].reshape(NUM_SUBCORES, 1, EDGES_F)
    dst_f = dst[:nf].reshape(NUM_SUBCORES, 1, EDGES_F)
    src_s = src[nf:].reshape(NUM_SUBCORES, 1, EDGES_S)
    dst_s = dst[nf:].reshape(NUM_SUBCORES, 1, EDGES_S)
    ident = jnp.arange(N_DEG // D, dtype=jnp.int32).reshape(1, N_DEG // D)

    deg2 = _degree_kernel(dst_f, dst_s, ident)
    h = _matmul(x, W, b.reshape(1, D))
    deg_col = deg2[0].reshape(N_DEG)[:N_NODES, None]
    g, r = _scale(deg_col, h)
    zeros = jnp.zeros((ROWS_PER_SUBCORE, D), jnp.float32)
    partials = _aggregate_kernel(g, src_f, dst_f, src_s, dst_s, zeros)
    return _combine(partials, r)


# speedup vs baseline: 1.3038x; 1.0962x over previous
import dataclasses
import functools

import jax
import jax.numpy as jnp
from jax import lax
from jax.experimental import pallas as pl
from jax.experimental.pallas import tpu as pltpu
from jax.experimental.pallas import tpu_sc as plsc

N_NODES = 10000
N_PAD = 10112
N_DEG = 10240
N_EDGES = 320000
D = 128

NUM_CORES = 2
NUM_SUBCORES = 16
NUM_TILES = NUM_CORES * NUM_SUBCORES
EDGES_PER_SUBCORE = N_EDGES // NUM_SUBCORES
EDGES_PER_TILE = N_EDGES // NUM_TILES
BATCH = 80
CHUNKS = EDGES_PER_TILE // BATCH
ROWS_PER_SUBCORE = N_PAD // NUM_SUBCORES
LANES = 16

_MESH = plsc.VectorSubcoreMesh(core_axis_name="c", subcore_axis_name="s")

_SC_PARAMS = pltpu.CompilerParams()
if "needs_layout_passes" in pltpu.CompilerParams.__dataclass_fields__:
    _SC_PARAMS = dataclasses.replace(_SC_PARAMS, needs_layout_passes=False)


@functools.partial(
    pl.kernel,
    out_type=jax.ShapeDtypeStruct((NUM_CORES, N_DEG // D, D), jnp.float32),
    mesh=_MESH,
    scratch_types=[
        pltpu.VMEM((EDGES_PER_SUBCORE,), jnp.int32),
        pltpu.VMEM((N_DEG // D, D), jnp.float32),
        pltpu.VMEM((1, N_DEG // D), jnp.int32),
        pltpu.VMEM_SHARED((N_DEG // D, D), jnp.float32),
    ],
    compiler_params=_SC_PARAMS,
)
def _degree_kernel(dst_hbm, ident_hbm, out_hbm, idx_v, deg_v, ident_v,
                   deg_shared):
    c = lax.axis_index("c")
    s = lax.axis_index("s")

    @pl.loop(0, N_DEG // D)
    def _(rr):
        @pl.loop(0, D // LANES)
        def _(cc):
            deg_v[rr, pl.ds(cc * LANES, LANES)] = jnp.zeros(
                (LANES,), jnp.float32)

    @pl.when(s < 5)
    def _():
        pltpu.sync_copy(deg_v.at[pl.ds(s * 16, 16)],
                        deg_shared.at[pl.ds(s * 16, 16)])

    pltpu.sync_copy(ident_hbm, ident_v)
    pltpu.sync_copy(dst_hbm.at[2 * s, 0],
                    idx_v.at[pl.ds(0, EDGES_PER_TILE)])
    pltpu.sync_copy(dst_hbm.at[2 * s + 1, 0],
                    idx_v.at[pl.ds(EDGES_PER_TILE, EDGES_PER_TILE)])
    ones = jnp.full((LANES,), 1.0, jnp.float32)

    @pl.loop(0, EDGES_PER_SUBCORE // LANES)
    def _(i):
        idx = idx_v[pl.ds(i * LANES, LANES)]
        plsc.addupdate_scatter(deg_v, [idx >> 7, idx & 127], ones)

    plsc.subcore_barrier()

    pltpu.sync_copy(deg_v, deg_shared.at[ident_v.at[0]], add=True)

    plsc.subcore_barrier()

    @pl.when(s < 5)
    def _():
        pltpu.sync_copy(deg_shared.at[pl.ds(s * 16, 16)],
                        out_hbm.at[c, pl.ds(s * 16, 16)])


@functools.partial(
    pl.kernel,
    out_type=jax.ShapeDtypeStruct((NUM_CORES, N_PAD, D), jnp.float32),
    mesh=_MESH,
    scratch_types=[
        pltpu.VMEM((EDGES_PER_TILE,), jnp.int32),
        pltpu.VMEM((EDGES_PER_TILE,), jnp.int32),
        pltpu.VMEM((BATCH, D), jnp.float32),
        pltpu.VMEM((BATCH, D), jnp.float32),
        pltpu.VMEM_SHARED((N_PAD, D), jnp.float32),
        pltpu.SemaphoreType.DMA,
        pltpu.SemaphoreType.DMA,
    ],
    compiler_params=_SC_PARAMS,
)
def _aggregate_kernel(g_hbm, src_hbm, dst_hbm, out_hbm,
                      src_v, dst_v, rows0_v, rows1_v, acc_shared,
                      sg0, sg1):
    c = lax.axis_index("c")
    s = lax.axis_index("s")
    wid = c * NUM_SUBCORES + s
    row0 = s * ROWS_PER_SUBCORE

    pltpu.sync_copy(g_hbm.at[pl.ds(row0, ROWS_PER_SUBCORE)],
                    acc_shared.at[pl.ds(row0, ROWS_PER_SUBCORE)])
    pltpu.sync_copy(src_hbm.at[wid, 0], src_v)
    pltpu.sync_copy(dst_hbm.at[wid, 0], dst_v)
    plsc.subcore_barrier()

    def gather(j, rows, sem):
        pltpu.async_copy(g_hbm.at[src_v.at[pl.ds(j * BATCH, BATCH)]],
                         rows, sem)

    def gather_wait(j, rows, sem):
        pltpu.make_async_copy(
            g_hbm.at[src_v.at[pl.ds(j * BATCH, BATCH)]], rows, sem).wait()

    def scatter(j, rows):
        pltpu.sync_copy(
            rows, acc_shared.at[dst_v.at[pl.ds(j * BATCH, BATCH)]],
            add=True)

    gather(0, rows0_v, sg0)

    @pl.loop(0, (CHUNKS - 1) // 2)
    def _(i):
        j = 2 * i
        gather(j + 1, rows1_v, sg1)
        gather_wait(j, rows0_v, sg0)
        scatter(j, rows0_v)
        gather(j + 2, rows0_v, sg0)
        gather_wait(j + 1, rows1_v, sg1)
        scatter(j + 1, rows1_v)

    gather_wait(CHUNKS - 1, rows0_v, sg0)
    scatter(CHUNKS - 1, rows0_v)

    plsc.subcore_barrier()
    pltpu.sync_copy(acc_shared.at[pl.ds(row0, ROWS_PER_SUBCORE)],
                    out_hbm.at[c, pl.ds(row0, ROWS_PER_SUBCORE)])


_BLOCK = 2000


def _linear_scale_body(deg_ref, x_ref, w_ref, b_ref, g_ref, r_ref):
    h = lax.dot_general(
        x_ref[...], w_ref[...], (((1,), (1,)), ((), ())),
        preferred_element_type=jnp.float32) + b_ref[...]
    r = lax.rsqrt(deg_ref[...] + 1.0)
    r_ref[...] = r
    g_ref[...] = h * r


def _linear_scale(deg_col, x, w, b2d):
    return pl.pallas_call(
        _linear_scale_body,
        grid=(N_NODES // _BLOCK,),
        in_specs=[
            pl.BlockSpec((_BLOCK, 1), lambda i: (i, 0)),
            pl.BlockSpec((_BLOCK, D), lambda i: (i, 0)),
            pl.BlockSpec((D, D), lambda i: (0, 0)),
            pl.BlockSpec((1, D), lambda i: (0, 0)),
        ],
        out_specs=[
            pl.BlockSpec((_BLOCK, D), lambda i: (i, 0)),
            pl.BlockSpec((_BLOCK, 1), lambda i: (i, 0)),
        ],
        out_shape=[
            jax.ShapeDtypeStruct((N_PAD, D), jnp.float32),
            jax.ShapeDtypeStruct((N_NODES, 1), jnp.float32),
        ],
    )(deg_col, x, w, b2d)


def _combine_body(p_ref, g_ref, r_ref, o_ref):
    o_ref[...] = (p_ref[0] + p_ref[1] - g_ref[...]) * r_ref[...]


def _combine(partials, g, r):
    return pl.pallas_call(
        _combine_body,
        grid=(N_NODES // _BLOCK,),
        in_specs=[
            pl.BlockSpec((NUM_CORES, _BLOCK, D), lambda i: (0, i, 0)),
            pl.BlockSpec((_BLOCK, D), lambda i: (i, 0)),
            pl.BlockSpec((_BLOCK, 1), lambda i: (i, 0)),
        ],
        out_specs=pl.BlockSpec((_BLOCK, D), lambda i: (i, 0)),
        out_shape=jax.ShapeDtypeStruct((N_NODES, D), jnp.float32),
    )(partials, g, r)


def kernel(x, edge_index, W, b):
    edges = edge_index.astype(jnp.int32)
    src_tiles = edges[0].reshape(NUM_TILES, 1, EDGES_PER_TILE)
    dst_tiles = edges[1].reshape(NUM_TILES, 1, EDGES_PER_TILE)
    ident = jnp.arange(N_DEG // D, dtype=jnp.int32).reshape(1, N_DEG // D)

    deg2 = _degree_kernel(dst_tiles, ident)
    deg_col = deg2[0].reshape(N_DEG)[:N_NODES, None]
    g, r = _linear_scale(deg_col, x, W, b.reshape(1, D))
    partials = _aggregate_kernel(g, src_tiles, dst_tiles)
    return _combine(partials, g, r)
